# baseline (speedup 1.0000x reference)
import functools

import jax
import jax.numpy as jnp
from jax import lax
from jax.experimental import pallas as pl
from jax.experimental.pallas import tpu as pltpu
from jax.experimental.pallas import tpu_sc as plsc

_D = 256
_K = 8192
_N = 8192
_NH = 2
_NTOK = _N // _NH
_TN = 1024
_NT = _NTOK // _TN
_CC = 0.25
_NW = 32
_B_PER_W = _NTOK // _NW
_CHUNK = 128


def _bsq_body(e_ref, bsq_ref):
    e = e_ref[...]
    bsq_ref[...] = jnp.sum(e * e, axis=0, keepdims=True)


_bsq_call = pl.pallas_call(
    _bsq_body,
    in_specs=[pl.BlockSpec((_D, _K), lambda: (0, 0))],
    out_specs=pl.BlockSpec((1, _K), lambda: (0, 0)),
    out_shape=jax.ShapeDtypeStruct((1, _K), jnp.float32),
)


def _dist_body(f_ref, e_hbm, bsq_ref, idx_ref, loss_ref,
               e_vmem, lacc_ref, e_sem):
    i = pl.program_id(0)
    f = f_ref[...]

    @pl.when(i == 0)
    def _init():
        lacc_ref[...] = jnp.zeros_like(lacc_ref)
        pltpu.make_async_copy(e_hbm, e_vmem, e_sem).start()
        pltpu.make_async_copy(e_hbm, e_vmem, e_sem).wait()

    m = jnp.dot(f, e_vmem[...], preferred_element_type=jnp.float32)
    a = jnp.sum(f * f, axis=1, keepdims=True)
    d = (a - 2.0 * m) + bsq_ref[...]
    minval = jnp.min(d, axis=1, keepdims=True)
    eq = d == minval
    iota = lax.broadcasted_iota(jnp.int32, (_TN, _K), 1).astype(jnp.float32)
    idxf = jnp.min(jnp.where(eq, iota, float(_K)), axis=1)
    idx_ref[0, 0, :] = idxf.astype(jnp.int32)
    lacc_ref[...] += jnp.sum(minval)[None, None]

    @pl.when(i == _NT - 1)
    def _fini():
        loss_ref[...] = lacc_ref[...]


_dist_call = pl.pallas_call(
    _dist_body,
    grid=(_NT,),
    in_specs=[
        pl.BlockSpec((_TN, _D), lambda i: (i, 0)),
        pl.BlockSpec(memory_space=pl.ANY),
        pl.BlockSpec((1, _K), lambda i: (0, 0)),
    ],
    out_specs=[
        pl.BlockSpec((1, 1, _TN), lambda i: (i, 0, 0)),
        pl.BlockSpec((1, 1), lambda i: (0, 0)),
    ],
    out_shape=[
        jax.ShapeDtypeStruct((_NT, 1, _TN), jnp.int32),
        jax.ShapeDtypeStruct((1, 1), jnp.float32),
    ],
    scratch_shapes=[
        pltpu.VMEM((_D, _K), jnp.float32),
        pltpu.VMEM((1, 1), jnp.float32),
        pltpu.SemaphoreType.DMA,
    ],
)


def _gather_body(w_hbm, idx_hbm, out_hbm, cnt_hbm,
                 idx_v, rows_v, ones_v, zero_v, tab_sh, sem):
    c = lax.axis_index("c")
    s = lax.axis_index("s")
    wid = s * 2 + c
    base = wid * _B_PER_W

    def _fill_ones(t, _):
        ones_v[pl.ds(t * 16, 16)] = jnp.ones((16,), jnp.float32)
        return _

    def _fill_zero(t, _):
        zero_v[pl.ds(t * 16, 16)] = jnp.zeros((16,), jnp.float32)
        return _

    lax.fori_loop(0, _CHUNK // 16, _fill_ones, 0)

    @pl.when(s == 0)
    def _zero_tab():
        lax.fori_loop(0, _K // 16, _fill_zero, 0)
        pltpu.sync_copy(zero_v, tab_sh)

    plsc.subcore_barrier()

    for j in range(_B_PER_W // _CHUNK):
        off = base + j * _CHUNK
        pltpu.sync_copy(idx_hbm.at[pl.ds(off, _CHUNK)], idx_v)
        cp = pltpu.async_copy(w_hbm.at[idx_v], rows_v, sem)
        pltpu.sync_copy(ones_v, tab_sh.at[idx_v], add=True)
        cp.wait()
        pltpu.sync_copy(rows_v, out_hbm.at[pl.ds(off, _CHUNK)])

    plsc.subcore_barrier()

    @pl.when(s == 0)
    def _emit_tab():
        pltpu.sync_copy(tab_sh, cnt_hbm.at[c])


@functools.lru_cache(maxsize=1)
def _gather_call():
    return functools.partial(
        pl.kernel,
        mesh=plsc.VectorSubcoreMesh(core_axis_name="c", subcore_axis_name="s"),
        out_type=[
            jax.ShapeDtypeStruct((_NTOK, _D), jnp.float32),
            jax.ShapeDtypeStruct((2, _K), jnp.float32),
        ],
        scratch_types=[
            pltpu.VMEM((_CHUNK,), jnp.int32),
            pltpu.VMEM((_CHUNK, _D), jnp.float32),
            pltpu.VMEM((_CHUNK,), jnp.float32),
            pltpu.VMEM((_K,), jnp.float32),
            pltpu.VMEM_SHARED((_K,), jnp.float32),
            pltpu.SemaphoreType.DMA,
        ],
    )(_gather_body)


def _fin_body(c1_ref, c2_ref, l1_ref, l2_ref, loss_ref, perp_ref):
    loss_ref[...] = (l1_ref[...] + l2_ref[...]) * ((1.0 + _CC) / float(_N * _D))
    c = jnp.sum(c1_ref[...] + c2_ref[...], axis=0, keepdims=True)
    p = c * (1.0 / float(_N))
    ent = jnp.sum(p * jnp.log(p + 1e-10), keepdims=True)
    perp_ref[...] = jnp.exp(-ent)


_fin_call = pl.pallas_call(
    _fin_body,
    in_specs=[
        pl.BlockSpec((2, _K), lambda: (0, 0)),
        pl.BlockSpec((2, _K), lambda: (0, 0)),
        pl.BlockSpec((1, 1), lambda: (0, 0)),
        pl.BlockSpec((1, 1), lambda: (0, 0)),
    ],
    out_specs=[
        pl.BlockSpec((1, 1), lambda: (0, 0)),
        pl.BlockSpec((1, 1), lambda: (0, 0)),
    ],
    out_shape=[
        jax.ShapeDtypeStruct((1, 1), jnp.float32),
        jax.ShapeDtypeStruct((1, 1), jnp.float32),
    ],
)


def kernel(x, embeddings):
    f = jnp.reshape(x, (_N, _D))
    bsq = _bsq_call(embeddings)
    w = jnp.transpose(embeddings)
    gather = _gather_call()
    halves = []
    for h in range(_NH):
        fh = lax.slice_in_dim(f, h * _NTOK, (h + 1) * _NTOK, axis=0)
        idx3, lacc = _dist_call(fh, embeddings, bsq)
        idx = jnp.reshape(idx3, (_NTOK,))
        quantized, cnt = gather(w, idx)
        halves.append((idx, quantized, cnt, lacc))
    loss, perp = _fin_call(halves[0][2], halves[1][2],
                           halves[0][3], halves[1][3])
    quantized_st = jnp.reshape(
        jnp.concatenate([h[1] for h in halves], axis=0), x.shape)
    encoding_indices = jnp.reshape(
        jnp.concatenate([h[0] for h in halves], axis=0), x.shape[:-1])
    return (quantized_st, jnp.reshape(loss, ()), jnp.reshape(perp, ()),
            encoding_indices)


# speedup vs baseline: 1.1196x; 1.1196x over previous
import functools

import jax
import jax.numpy as jnp
from jax import lax
from jax.experimental import pallas as pl
from jax.experimental.pallas import tpu as pltpu
from jax.experimental.pallas import tpu_sc as plsc

_D = 256
_K = 8192
_TN = 1024
_NT = 8
_N = _NT * _TN
_CC = 0.25
_NW = 32
_B_PER_W = _N // _NW
_CHUNK = 128
_NKC = 4


def _bsq_body(e_ref, bsq_ref):
    e = e_ref[...]
    bsq_ref[...] = jnp.sum(e * e, axis=0, keepdims=True)


_bsq_call = pl.pallas_call(
    _bsq_body,
    in_specs=[pl.BlockSpec((_D, _K), lambda: (0, 0))],
    out_specs=pl.BlockSpec((1, _K), lambda: (0, 0)),
    out_shape=jax.ShapeDtypeStruct((1, _K), jnp.float32),
)


def _dist_body(f_ref, e_hbm, bsq_ref, idx_ref, loss_ref,
               e_vmem, lacc_ref, e_sem):
    i = pl.program_id(0)
    f = f_ref[...]

    @pl.when(i == 0)
    def _init():
        lacc_ref[...] = jnp.zeros_like(lacc_ref)
        pltpu.make_async_copy(e_hbm, e_vmem, e_sem).start()
        pltpu.make_async_copy(e_hbm, e_vmem, e_sem).wait()

    a = jnp.sum(f * f, axis=1, keepdims=True)
    _KC = _K // _NKC
    iota = lax.broadcasted_iota(jnp.int32, (_TN, _KC), 1).astype(jnp.float32)
    minvs, idxfs = [], []
    for c in range(_NKC):
        ec = e_vmem[:, pl.ds(c * _KC, _KC)]
        bc = bsq_ref[:, pl.ds(c * _KC, _KC)]
        mc = jnp.dot(f, ec, preferred_element_type=jnp.float32)
        dc = (a - 2.0 * mc) + bc
        mv = jnp.min(dc, axis=1, keepdims=True)
        eqc = dc == mv
        idc = jnp.min(jnp.where(eqc, iota, float(_KC)),
                      axis=1, keepdims=True) + float(c * _KC)
        minvs.append(mv)
        idxfs.append(idc)
    minval = minvs[0]
    for mv in minvs[1:]:
        minval = jnp.minimum(minval, mv)
    idxf = jnp.full_like(idxfs[0], float(_K))
    for mv, idc in zip(minvs, idxfs):
        idxf = jnp.minimum(idxf, jnp.where(mv == minval, idc, float(_K)))
    idx_ref[0, 0, :] = idxf[:, 0].astype(jnp.int32)
    lacc_ref[...] += jnp.sum(minval)[None, None]

    @pl.when(i == _NT - 1)
    def _fini():
        loss_ref[...] = lacc_ref[...] * ((1.0 + _CC) / float(_N * _D))


_dist_call = pl.pallas_call(
    _dist_body,
    grid=(_NT,),
    in_specs=[
        pl.BlockSpec((_TN, _D), lambda i: (i, 0)),
        pl.BlockSpec(memory_space=pl.ANY),
        pl.BlockSpec((1, _K), lambda i: (0, 0)),
    ],
    out_specs=[
        pl.BlockSpec((1, 1, _TN), lambda i: (i, 0, 0)),
        pl.BlockSpec((1, 1), lambda i: (0, 0)),
    ],
    out_shape=[
        jax.ShapeDtypeStruct((_NT, 1, _TN), jnp.int32),
        jax.ShapeDtypeStruct((1, 1), jnp.float32),
    ],
    scratch_shapes=[
        pltpu.VMEM((_D, _K), jnp.float32),
        pltpu.VMEM((1, 1), jnp.float32),
        pltpu.SemaphoreType.DMA,
    ],
)


def _gather_body(w_hbm, idx_hbm, out_hbm, cnt_hbm,
                 idx_v, rows_v, ones_v, zero_v, tab_sh, sem):
    c = lax.axis_index("c")
    s = lax.axis_index("s")
    wid = s * 2 + c
    base = wid * _B_PER_W

    def _fill_ones(t, _):
        ones_v[pl.ds(t * 16, 16)] = jnp.ones((16,), jnp.float32)
        return _

    def _fill_zero(t, _):
        zero_v[pl.ds(t * 16, 16)] = jnp.zeros((16,), jnp.float32)
        return _

    lax.fori_loop(0, _CHUNK // 16, _fill_ones, 0)

    @pl.when(s == 0)
    def _zero_tab():
        lax.fori_loop(0, _K // 16, _fill_zero, 0)
        pltpu.sync_copy(zero_v, tab_sh)

    plsc.subcore_barrier()

    for j in range(_B_PER_W // _CHUNK):
        off = base + j * _CHUNK
        pltpu.sync_copy(idx_hbm.at[pl.ds(off, _CHUNK)], idx_v)
        cp = pltpu.async_copy(w_hbm.at[idx_v], rows_v, sem)
        pltpu.sync_copy(ones_v, tab_sh.at[idx_v], add=True)
        cp.wait()
        pltpu.sync_copy(rows_v, out_hbm.at[pl.ds(off, _CHUNK)])

    plsc.subcore_barrier()

    @pl.when(s == 0)
    def _emit_tab():
        pltpu.sync_copy(tab_sh, cnt_hbm.at[c])


@functools.lru_cache(maxsize=1)
def _gather_call():
    return functools.partial(
        pl.kernel,
        mesh=plsc.VectorSubcoreMesh(core_axis_name="c", subcore_axis_name="s"),
        out_type=[
            jax.ShapeDtypeStruct((_N, _D), jnp.float32),
            jax.ShapeDtypeStruct((2, _K), jnp.float32),
        ],
        scratch_types=[
            pltpu.VMEM((_CHUNK,), jnp.int32),
            pltpu.VMEM((_CHUNK, _D), jnp.float32),
            pltpu.VMEM((_CHUNK,), jnp.float32),
            pltpu.VMEM((_K,), jnp.float32),
            pltpu.VMEM_SHARED((_K,), jnp.float32),
            pltpu.SemaphoreType.DMA,
        ],
    )(_gather_body)


def _perp_body(cnt_ref, perp_ref):
    c = jnp.sum(cnt_ref[...], axis=0, keepdims=True)
    p = c * (1.0 / float(_N))
    ent = jnp.sum(p * jnp.log(p + 1e-10), keepdims=True)
    perp_ref[...] = jnp.exp(-ent)


_perp_call = pl.pallas_call(
    _perp_body,
    in_specs=[pl.BlockSpec((2, _K), lambda: (0, 0))],
    out_specs=pl.BlockSpec((1, 1), lambda: (0, 0)),
    out_shape=jax.ShapeDtypeStruct((1, 1), jnp.float32),
)


def kernel(x, embeddings):
    f = jnp.reshape(x, (_N, _D))
    bsq = _bsq_call(embeddings)
    idx3, loss = _dist_call(f, embeddings, bsq)
    idx = jnp.reshape(idx3, (_N,))
    w = jnp.transpose(embeddings)
    quantized, cnt = _gather_call()(w, idx)
    perp = _perp_call(cnt)
    quantized_st = jnp.reshape(quantized, x.shape)
    encoding_indices = jnp.reshape(idx, x.shape[:-1])
    return (quantized_st, jnp.reshape(loss, ()), jnp.reshape(perp, ()),
            encoding_indices)


# speedup vs baseline: 1.1782x; 1.0523x over previous
import functools

import jax
import jax.numpy as jnp
from jax import lax
from jax.experimental import pallas as pl
from jax.experimental.pallas import tpu as pltpu
from jax.experimental.pallas import tpu_sc as plsc

_D = 256
_K = 8192
_TN = 1024
_NT = 8
_N = _NT * _TN
_CC = 0.25
_NW = 32
_B_PER_W = _N // _NW
_CHUNK = 128


def _bsq_body(e_ref, bsq_ref):
    e = e_ref[...]
    bsq_ref[...] = jnp.sum(e * e, axis=0, keepdims=True)


_bsq_call = pl.pallas_call(
    _bsq_body,
    in_specs=[pl.BlockSpec((_D, _K), lambda: (0, 0))],
    out_specs=pl.BlockSpec((1, _K), lambda: (0, 0)),
    out_shape=jax.ShapeDtypeStruct((1, _K), jnp.float32),
)


def _dist_body(f_ref, e_hbm, bsq_ref, idx_ref, loss_ref,
               e_vmem, lacc_ref, e_sem):
    i = pl.program_id(0)
    f = f_ref[...]

    @pl.when(i == 0)
    def _init():
        lacc_ref[...] = jnp.zeros_like(lacc_ref)
        pltpu.make_async_copy(e_hbm, e_vmem, e_sem).start()
        pltpu.make_async_copy(e_hbm, e_vmem, e_sem).wait()

    m = jnp.dot(f, e_vmem[...], preferred_element_type=jnp.float32)
    a = jnp.sum(f * f, axis=1, keepdims=True)
    d = (a - 2.0 * m) + bsq_ref[...]
    minval = jnp.min(d, axis=1, keepdims=True)
    eq = d == minval
    iota = lax.broadcasted_iota(jnp.int32, (_TN, _K), 1).astype(jnp.float32)
    idxf = jnp.min(jnp.where(eq, iota, float(_K)), axis=1)
    idx_ref[0, 0, :] = idxf.astype(jnp.int32)
    lacc_ref[...] += jnp.sum(minval)[None, None]

    @pl.when(i == _NT - 1)
    def _fini():
        loss_ref[...] = lacc_ref[...] * ((1.0 + _CC) / float(_N * _D))


_dist_call = pl.pallas_call(
    _dist_body,
    grid=(_NT,),
    in_specs=[
        pl.BlockSpec((_TN, _D), lambda i: (i, 0)),
        pl.BlockSpec(memory_space=pl.ANY),
        pl.BlockSpec((1, _K), lambda i: (0, 0)),
    ],
    out_specs=[
        pl.BlockSpec((1, 1, _TN), lambda i: (i, 0, 0)),
        pl.BlockSpec((1, 1), lambda i: (0, 0)),
    ],
    out_shape=[
        jax.ShapeDtypeStruct((_NT, 1, _TN), jnp.int32),
        jax.ShapeDtypeStruct((1, 1), jnp.float32),
    ],
    scratch_shapes=[
        pltpu.VMEM((_D, _K), jnp.float32),
        pltpu.VMEM((1, 1), jnp.float32),
        pltpu.SemaphoreType.DMA,
    ],
)


def _gather_body(w_hbm, idx_hbm, out_hbm, cnt_hbm,
                 idx_v, rows_v, ones_v, zero_v, tab_sh, sem):
    c = lax.axis_index("c")
    s = lax.axis_index("s")
    wid = s * 2 + c
    base = wid * _B_PER_W

    def _fill_ones(t, _):
        ones_v[pl.ds(t * 16, 16)] = jnp.ones((16,), jnp.float32)
        return _

    def _fill_zero(t, _):
        zero_v[pl.ds(t * 16, 16)] = jnp.zeros((16,), jnp.float32)
        return _

    lax.fori_loop(0, _CHUNK // 16, _fill_ones, 0)

    @pl.when(s == 0)
    def _zero_tab():
        lax.fori_loop(0, _K // 16, _fill_zero, 0)
        pltpu.sync_copy(zero_v, tab_sh)

    plsc.subcore_barrier()

    for j in range(_B_PER_W // _CHUNK):
        off = base + j * _CHUNK
        pltpu.sync_copy(idx_hbm.at[pl.ds(off, _CHUNK)], idx_v)
        cp = pltpu.async_copy(w_hbm.at[idx_v], rows_v, sem)
        pltpu.sync_copy(ones_v, tab_sh.at[idx_v], add=True)
        cp.wait()
        pltpu.sync_copy(rows_v, out_hbm.at[pl.ds(off, _CHUNK)])

    plsc.subcore_barrier()

    @pl.when(s == 0)
    def _emit_tab():
        pltpu.sync_copy(tab_sh, cnt_hbm.at[c])


@functools.lru_cache(maxsize=1)
def _gather_call():
    return functools.partial(
        pl.kernel,
        mesh=plsc.VectorSubcoreMesh(core_axis_name="c", subcore_axis_name="s"),
        out_type=[
            jax.ShapeDtypeStruct((_N, _D), jnp.float32),
            jax.ShapeDtypeStruct((2, _K), jnp.float32),
        ],
        scratch_types=[
            pltpu.VMEM((_CHUNK,), jnp.int32),
            pltpu.VMEM((_CHUNK, _D), jnp.float32),
            pltpu.VMEM((_CHUNK,), jnp.float32),
            pltpu.VMEM((_K,), jnp.float32),
            pltpu.VMEM_SHARED((_K,), jnp.float32),
            pltpu.SemaphoreType.DMA,
        ],
    )(_gather_body)


def _perp_body(cnt_ref, perp_ref):
    c = jnp.sum(cnt_ref[...], axis=0, keepdims=True)
    p = c * (1.0 / float(_N))
    ent = jnp.sum(p * jnp.log(p + 1e-10), keepdims=True)
    perp_ref[...] = jnp.exp(-ent)


_perp_call = pl.pallas_call(
    _perp_body,
    in_specs=[pl.BlockSpec((2, _K), lambda: (0, 0))],
    out_specs=pl.BlockSpec((1, 1), lambda: (0, 0)),
    out_shape=jax.ShapeDtypeStruct((1, 1), jnp.float32),
)


def kernel(x, embeddings):
    f = jnp.reshape(x, (_N, _D))
    bsq = _bsq_call(embeddings)
    idx3, loss = _dist_call(f, embeddings, bsq)
    idx = jnp.reshape(idx3, (_N,))
    w = jnp.transpose(embeddings)
    quantized, cnt = _gather_call()(w, idx)
    perp = _perp_call(cnt)
    quantized_st = jnp.reshape(quantized, x.shape)
    encoding_indices = jnp.reshape(idx, x.shape[:-1])
    return (quantized_st, jnp.reshape(loss, ()), jnp.reshape(perp, ()),
            encoding_indices)
